# baseline (speedup 1.0000x reference)
import functools

import jax
import jax.numpy as jnp
from jax import lax
from jax.experimental import pallas as pl
from jax.experimental.pallas import tpu as pltpu
from jax.experimental.pallas import tpu_sc as plsc

_B = 16384
_D = 32
_CHUNK = 128


@functools.cache
def _build_gather():
    info = plsc.get_sparse_core_info()
    nc, ns = info.num_cores, info.num_subcores
    nw = nc * ns
    b_per_w = _B // nw
    n_chunks = b_per_w // _CHUNK
    mesh = plsc.VectorSubcoreMesh(core_axis_name="c", subcore_axis_name="s")

    @functools.partial(
        pl.kernel,
        mesh=mesh,
        compiler_params=pltpu.CompilerParams(use_tc_tiling_on_sc=False),
        out_type=jax.ShapeDtypeStruct((_B, _D), jnp.float32),
        scratch_types=[
            pltpu.VMEM((n_chunks, _CHUNK), jnp.int32),
            pltpu.VMEM((b_per_w, _D), jnp.float32),
            pltpu.SemaphoreType.DMA,
        ],
    )
    def gather(table_hbm, idx_hbm, out_hbm, idx_v, rows_v, sem):
        wid = lax.axis_index("s") * nc + lax.axis_index("c")
        pltpu.sync_copy(idx_hbm.at[wid], idx_v)
        copies = [
            pltpu.async_copy(
                table_hbm.at[idx_v.at[j]],
                rows_v.at[pl.ds(j * _CHUNK, _CHUNK)],
                sem,
            )
            for j in range(n_chunks)
        ]
        for c in copies:
            c.wait()
        pltpu.sync_copy(rows_v, out_hbm.at[pl.ds(wid * b_per_w, b_per_w)])

    return gather, nw, n_chunks


def kernel(condition, emb_table):
    gather, nw, n_chunks = _build_gather()
    idx = condition.astype(jnp.int32).reshape(nw, n_chunks, _CHUNK)
    return gather(emb_table, idx)


# speedup vs baseline: 4.2727x; 4.2727x over previous
import functools

import jax
import jax.numpy as jnp
from jax import lax
from jax.experimental import pallas as pl
from jax.experimental.pallas import tpu as pltpu
from jax.experimental.pallas import tpu_sc as plsc

_B = 16384
_V = 1000000
_D = 32
_LANES = 128
_TAIL_START = (_V // _LANES) * _LANES
_LAST_SLAB = _TAIL_START - _LANES
_NBUF = 8


@functools.cache
def _build_gather():
    info = plsc.get_sparse_core_info()
    nc, ns = info.num_cores, info.num_subcores
    nw = nc * ns
    bpw = _B // nw
    mesh = plsc.VectorSubcoreMesh(core_axis_name="c", subcore_axis_name="s")

    @functools.partial(
        pl.kernel,
        mesh=mesh,
        compiler_params=pltpu.CompilerParams(needs_layout_passes=False),
        out_type=jax.ShapeDtypeStruct((_D, _B), jnp.float32),
        scratch_types=[
            pltpu.VMEM((bpw + 16,), jnp.int32),
            pltpu.VMEM((_NBUF, _D, _LANES), jnp.float32),
            pltpu.VMEM((_D, _V - _TAIL_START), jnp.float32),
            pltpu.VMEM((_D, bpw), jnp.float32),
        ]
        + [pltpu.SemaphoreType.DMA] * _NBUF,
    )
    def gather(table_t, idx_hbm, tail_t, out_t, idx_v, slabs, tail_v,
               cols, *sems):
        wid = lax.axis_index("s") * nc + lax.axis_index("c")
        base = wid * bpw
        pltpu.sync_copy(idx_hbm.at[pl.ds(base, bpw)], idx_v.at[pl.ds(0, bpw)])

        def idx_at(i):
            return idx_v[pl.ds(i, 16)][0]
        pltpu.sync_copy(tail_t, tail_v)

        d0 = lax.iota(jnp.int32, 16)
        d1 = d0 + 16

        def slab_start(i):
            c = idx_at(i)
            t = jnp.minimum((c >> 7) << 7, _LAST_SLAB)
            return pl.multiple_of(t, _LANES)

        def fire(i, b):
            pltpu.async_copy(
                table_t.at[:, pl.ds(slab_start(i), _LANES)],
                slabs.at[b],
                sems[b],
            )

        def extract(i, b):
            c = idx_at(i)
            t = jnp.minimum((c >> 7) << 7, _LAST_SLAB)
            l = jnp.minimum(c - t, _LANES - 1)
            lvec = jnp.full((16,), l, jnp.int32)
            ltail = jnp.full(
                (16,),
                jnp.clip(c - _TAIL_START, 0, _V - _TAIL_START - 1),
                jnp.int32,
            )
            use_tail = jnp.full((16,), c >= _TAIL_START, jnp.bool_)
            ivec = jnp.full((16,), i, jnp.int32)
            for dvec in (d0, d1):
                v = plsc.load_gather(slabs.at[b], [dvec, lvec])
                u = plsc.load_gather(tail_v, [dvec, ltail])
                plsc.store_scatter(
                    cols, [dvec, ivec], jnp.where(use_tail, u, v)
                )

        for b in range(_NBUF):
            fire(b, b)

        def steady(j, carry):
            i = j * _NBUF
            for b in range(_NBUF):
                pltpu.make_async_copy(
                    table_t.at[:, pl.ds(0, _LANES)], slabs.at[b], sems[b]
                ).wait()
                extract(i + b, b)

                @pl.when(i + b + _NBUF < bpw)
                def _():
                    fire(i + b + _NBUF, b)

            return carry

        lax.fori_loop(0, bpw // _NBUF, steady, 0)
        pltpu.sync_copy(cols, out_t.at[:, pl.ds(base, bpw)])

    return gather, nw


def kernel(condition, emb_table):
    gather, _ = _build_gather()
    table_t = emb_table.T
    tail_t = table_t[:, _TAIL_START:]
    idx = condition.astype(jnp.int32)
    return gather(table_t, idx, tail_t).T
